# baseline (speedup 1.0000x reference)
import jax
import jax.numpy as jnp
from jax import lax
from jax.experimental import pallas as pl
from jax.experimental.pallas import tpu as pltpu, tpu_sc as plsc

B = 16384
K = 64
NC = 2
NS = 16
L = 16
NW = NC * NS
BPW = B // NW
CHUNK = 128
NCHUNK = BPW // CHUNK
GROUPS = BPW // L


def _body(user_hbm, item_hbm, uemb_hbm, iemb_hbm, out_hbm,
          idx_u, idx_i, rows_u, rows_i, out_v, sem):
    wid = lax.axis_index("s") * NC + lax.axis_index("c")
    base = wid * BPW

    pltpu.sync_copy(user_hbm.at[pl.ds(base, BPW)], idx_u)
    pltpu.sync_copy(item_hbm.at[pl.ds(base, BPW)], idx_i)

    copies = []
    for j in range(NCHUNK):
        sl = pl.ds(j * CHUNK, CHUNK)
        copies.append(pltpu.async_copy(uemb_hbm.at[idx_u.at[sl]], rows_u.at[sl], sem))
        copies.append(pltpu.async_copy(iemb_hbm.at[idx_i.at[sl]], rows_i.at[sl], sem))
    for c in copies:
        c.wait()

    iota = lax.iota(jnp.int32, L)

    def group(g, carry):
        rows = g * L + iota
        acc = jnp.zeros((L,), jnp.float32)
        for k in range(K):
            cols = (iota + k) & (K - 1)
            u = plsc.load_gather(rows_u, [rows, cols])
            v = plsc.load_gather(rows_i, [rows, cols])
            acc = acc + u * v
        out_v[pl.ds(g * L, L)] = acc
        return carry

    lax.fori_loop(0, GROUPS, group, 0)
    pltpu.sync_copy(out_v, out_hbm.at[pl.ds(base, BPW)])


def kernel(user, item, user_emb, item_emb):
    mesh = plsc.VectorSubcoreMesh(
        core_axis_name="c", subcore_axis_name="s",
        num_cores=NC, num_subcores=NS)
    f = pl.kernel(
        _body,
        out_type=jax.ShapeDtypeStruct((B,), jnp.float32),
        mesh=mesh,
        scratch_types=[
            pltpu.VMEM((BPW,), jnp.int32),
            pltpu.VMEM((BPW,), jnp.int32),
            pltpu.VMEM((BPW, K), jnp.float32),
            pltpu.VMEM((BPW, K), jnp.float32),
            pltpu.VMEM((BPW,), jnp.float32),
            pltpu.SemaphoreType.DMA,
        ],
        compiler_params=pltpu.CompilerParams(
            needs_layout_passes=False, use_tc_tiling_on_sc=False),
    )
    return f(user, item, user_emb, item_emb)


# speedup vs baseline: 2.3959x; 2.3959x over previous
import jax
import jax.numpy as jnp
from jax import lax
from jax.experimental import pallas as pl
from jax.experimental.pallas import tpu as pltpu, tpu_sc as plsc

B = 16384
K = 64
NC = 2
NS = 16
L = 16
NW = NC * NS
BPW = B // NW
NB = 2
ROUNDS = BPW // NB


def _body(user_hbm, item_hbm, uet_hbm, iet_hbm, out_hbm,
          idx_uv, idx_iv, out_vv,
          ua0, ua1, ia0, ia1, ub0, ub1, ib0, ib1,
          sem_a, sem_b):
    wid = lax.axis_index("s") * NC + lax.axis_index("c")
    base = wid * BPW

    pltpu.sync_copy(user_hbm.at[pl.ds(base, BPW)], idx_uv)
    pltpu.sync_copy(item_hbm.at[pl.ds(base, BPW)], idx_iv)

    iota = lax.iota(jnp.int32, L)

    def scalar_at(ref_v, j):
        v = ref_v[pl.ds((j >> 4) * L, L)]
        return jnp.sum(jnp.where(iota == (j & (L - 1)), v, 0))

    def enqueue(j, ubuf, ibuf, sem):
        ru = scalar_at(idx_uv, j)
        ri = scalar_at(idx_iv, j)
        ou = pl.multiple_of((ru >> 7) << 7, 128)
        oi = pl.multiple_of((ri >> 7) << 7, 128)
        pltpu.async_copy(uet_hbm.at[:, pl.ds(ou, 128)], ubuf, sem)
        pltpu.async_copy(iet_hbm.at[:, pl.ds(oi, 128)], ibuf, sem)

    def drain4(sem):
        for _ in range(2 * NB):
            pltpu.make_async_copy(
                uet_hbm.at[:, pl.ds(0, 128)], ua0, sem).wait()

    def extract(j, ubuf, ibuf, acc_out):
        cu = jnp.full((L,), scalar_at(idx_uv, j) & 127, jnp.int32)
        ci = jnp.full((L,), scalar_at(idx_iv, j) & 127, jnp.int32)
        acc = jnp.zeros((L,), jnp.float32)
        for m in range(K // L):
            rows = m * L + iota
            u = plsc.load_gather(ubuf, [rows, cu])
            v = plsc.load_gather(ibuf, [rows, ci])
            acc = acc + u * v
        dot = jnp.sum(acc)
        return jnp.where(iota == (j & (L - 1)), dot, acc_out)

    enqueue(0, ua0, ia0, sem_a)
    enqueue(1, ua1, ia1, sem_a)

    def dround(rr, acc_out):
        j0 = 4 * rr
        enqueue(j0 + 2, ub0, ib0, sem_b)
        enqueue(j0 + 3, ub1, ib1, sem_b)
        drain4(sem_a)
        acc_out = extract(j0, ua0, ia0, acc_out)
        acc_out = extract(j0 + 1, ua1, ia1, acc_out)

        @pl.when(rr < ROUNDS // 2 - 1)
        def _():
            enqueue(j0 + 4, ua0, ia0, sem_a)
            enqueue(j0 + 5, ua1, ia1, sem_a)

        drain4(sem_b)
        acc_out = extract(j0 + 2, ub0, ib0, acc_out)
        acc_out = extract(j0 + 3, ub1, ib1, acc_out)

        @pl.when((rr & 3) == 3)
        def _():
            out_vv[pl.ds((j0 >> 4) * L, L)] = acc_out

        return jnp.where((rr & 3) == 3, jnp.zeros((L,), jnp.float32), acc_out)

    lax.fori_loop(0, ROUNDS // 2, dround, jnp.zeros((L,), jnp.float32))
    pltpu.sync_copy(out_vv, out_hbm.at[pl.ds(base, BPW)])


def kernel(user, item, user_emb, item_emb):
    mesh = plsc.VectorSubcoreMesh(
        core_axis_name="c", subcore_axis_name="s",
        num_cores=NC, num_subcores=NS)
    blk = lambda: pltpu.VMEM((K, 128), jnp.float32)
    f = pl.kernel(
        _body,
        out_type=jax.ShapeDtypeStruct((B,), jnp.float32),
        mesh=mesh,
        scratch_types=[
            pltpu.VMEM((BPW,), jnp.int32),
            pltpu.VMEM((BPW,), jnp.int32),
            pltpu.VMEM((BPW,), jnp.float32),
            blk(), blk(), blk(), blk(), blk(), blk(), blk(), blk(),
            pltpu.SemaphoreType.DMA,
            pltpu.SemaphoreType.DMA,
        ],
        compiler_params=pltpu.CompilerParams(
            needs_layout_passes=False, use_tc_tiling_on_sc=True),
    )
    return f(user, item, user_emb.T, item_emb.T)


# speedup vs baseline: 2.5706x; 1.0729x over previous
import jax
import jax.numpy as jnp
from jax import lax
from jax.experimental import pallas as pl
from jax.experimental.pallas import tpu as pltpu, tpu_sc as plsc

B = 16384
K = 64
NC = 2
NS = 16
L = 16
NW = NC * NS
BPW = B // NW
NB = 2
ROUNDS = BPW // NB


def _body(user_hbm, item_hbm, uet_hbm, iet_hbm, out_hbm,
          idx_uv, idx_iv, out_vv, accr,
          ua0, ua1, ia0, ia1,
          ub0, ub1, ib0, ib1,
          uc0, uc1, ic0, ic1,
          sem_a, sem_b, sem_c):
    wid = lax.axis_index("s") * NC + lax.axis_index("c")
    base = wid * BPW

    pltpu.sync_copy(user_hbm.at[pl.ds(base, BPW)], idx_uv)
    pltpu.sync_copy(item_hbm.at[pl.ds(base, BPW)], idx_iv)

    iota = lax.iota(jnp.int32, L)
    sets = ((ua0, ua1, ia0, ia1, sem_a),
            (ub0, ub1, ib0, ib1, sem_b),
            (uc0, uc1, ic0, ic1, sem_c))

    def scalar_at(ref_v, j):
        v = ref_v[pl.ds((j >> 4) * L, L)]
        return jnp.sum(jnp.where(iota == (j & (L - 1)), v, 0))

    def enqueue_round(r, st):
        u0, u1, i0, i1, sem = st
        for q, (ub, ib) in enumerate(((u0, i0), (u1, i1))):
            j = NB * r + q
            ou = pl.multiple_of((scalar_at(idx_uv, j) >> 7) << 7, 128)
            oi = pl.multiple_of((scalar_at(idx_iv, j) >> 7) << 7, 128)
            pltpu.async_copy(uet_hbm.at[:, pl.ds(ou, 128)], ub, sem)
            pltpu.async_copy(iet_hbm.at[:, pl.ds(oi, 128)], ib, sem)

    def drain_round(st):
        for _ in range(2 * NB):
            pltpu.make_async_copy(
                uet_hbm.at[:, pl.ds(0, 128)], ua0, st[4]).wait()

    def extract(j, ubuf, ibuf):
        cu = jnp.full((L,), scalar_at(idx_uv, j) & 127, jnp.int32)
        ci = jnp.full((L,), scalar_at(idx_iv, j) & 127, jnp.int32)
        acc = jnp.zeros((L,), jnp.float32)
        for m in range(K // L):
            rows = m * L + iota
            acc = acc + plsc.load_gather(ubuf, [rows, cu]) * \
                plsc.load_gather(ibuf, [rows, ci])
        accr[...] = jnp.where(iota == (j & (L - 1)), jnp.sum(acc), accr[...])

    enqueue_round(0, sets[0])
    enqueue_round(1, sets[1])

    def rbody(r, carry):
        for s in range(3):
            @pl.when(r % 3 == s)
            def _(s=s):
                @pl.when(r + 2 < ROUNDS)
                def _():
                    enqueue_round(r + 2, sets[(s + 2) % 3])

                drain_round(sets[s])
                extract(NB * r, sets[s][0], sets[s][2])
                extract(NB * r + 1, sets[s][1], sets[s][3])

        @pl.when((r & 7) == 7)
        def _():
            out_vv[pl.ds(((NB * r) >> 4) * L, L)] = accr[...]
            accr[...] = jnp.zeros((L,), jnp.float32)

        return carry

    accr[...] = jnp.zeros((L,), jnp.float32)
    lax.fori_loop(0, ROUNDS, rbody, 0)
    pltpu.sync_copy(out_vv, out_hbm.at[pl.ds(base, BPW)])


def kernel(user, item, user_emb, item_emb):
    mesh = plsc.VectorSubcoreMesh(
        core_axis_name="c", subcore_axis_name="s",
        num_cores=NC, num_subcores=NS)
    blk = lambda: pltpu.VMEM((K, 128), jnp.float32)
    f = pl.kernel(
        _body,
        out_type=jax.ShapeDtypeStruct((B,), jnp.float32),
        mesh=mesh,
        scratch_types=[
            pltpu.VMEM((BPW,), jnp.int32),
            pltpu.VMEM((BPW,), jnp.int32),
            pltpu.VMEM((BPW,), jnp.float32),
            pltpu.VMEM((L,), jnp.float32),
            blk(), blk(), blk(), blk(),
            blk(), blk(), blk(), blk(),
            blk(), blk(), blk(), blk(),
            pltpu.SemaphoreType.DMA,
            pltpu.SemaphoreType.DMA,
            pltpu.SemaphoreType.DMA,
        ],
        compiler_params=pltpu.CompilerParams(
            needs_layout_passes=False, use_tc_tiling_on_sc=True),
    )
    return f(user, item, user_emb.T, item_emb.T)
